# baseline (speedup 1.0000x reference)
import jax
import jax.numpy as jnp
from jax.experimental import pallas as pl
from jax.experimental.pallas import tpu as pltpu

B = 4096
D = 2048
H = 1024
C = 26
K_NEIGHBORS = 8
BM = 512


def _graph_kernel(ce_ref, wqk_ref, bq_ref, bk_ref,
                  wg1_ref, bg1_ref, wg2_ref, bg2_ref, h_ref):
    ce = ce_ref[:]
    qk = jnp.dot(ce, wqk_ref[:], preferred_element_type=jnp.float32)
    q = qk[:, 0:256] + bq_ref[:]
    kk = qk[:, 256:512] + bk_ref[:]
    scores = jax.lax.dot_general(
        q, kk, (((1,), (1,)), ((), ())),
        preferred_element_type=jnp.float32)

    col = jax.lax.broadcasted_iota(jnp.int32, (C, C), 1)
    big = jnp.int32(2 * C)
    work = scores
    sel = jnp.zeros((C, C), jnp.float32)
    for _ in range(K_NEIGHBORS):
        m = jnp.max(work, axis=1, keepdims=True)
        is_max = work >= m
        first = jnp.min(jnp.where(is_max, col, big), axis=1, keepdims=True)
        sel = sel + (col == first).astype(jnp.float32)
        work = jnp.where(col == first, -jnp.inf, work)

    eyef = (jax.lax.broadcasted_iota(jnp.int32, (C, C), 0) == col)
    eyef = eyef.astype(jnp.float32)
    mo = sel * (1.0 - eyef)
    deg = jnp.sum(mo, axis=0) + 1.0
    dinv = jax.lax.rsqrt(deg)

    def norm_matmul(y):
        yd = y * dinv[:, None]
        moty = jax.lax.dot_general(
            mo, yd, (((0,), (0,)), ((), ())),
            preferred_element_type=jnp.float32)
        return (moty + yd) * dinv[:, None]

    g1 = norm_matmul(
        jnp.dot(ce, wg1_ref[:], preferred_element_type=jnp.float32))
    g1 = jnp.maximum(g1 + bg1_ref[:], 0.0)
    h = norm_matmul(
        jnp.dot(g1, wg2_ref[:], preferred_element_type=jnp.float32))
    h_ref[:] = h + bg2_ref[:]


def _big_kernel(x_ref, w1_ref, b1_ref, w2_ref, b2_ref,
                wp_ref, bp_ref, h_ref, out_ref):
    xb = x_ref[:]
    u = jnp.dot(xb, w1_ref[:], preferred_element_type=jnp.float32)
    u = jnp.maximum(u + b1_ref[:], 0.0)
    p = jnp.dot(xb, wp_ref[:], preferred_element_type=jnp.float32)
    p = jnp.maximum(p + bp_ref[:], 0.0)
    cnn = jnp.dot(u, w2_ref[:], preferred_element_type=jnp.float32)
    gcn = jax.lax.dot_general(
        p, h_ref[:], (((1,), (1,)), ((), ())),
        preferred_element_type=jnp.float32)
    out_ref[:] = cnn + gcn + b2_ref[:]


@jax.jit
def kernel(x, W_i2c_1, b_i2c_1, W_i2c_2, b_i2c_2, W_proj, b_proj,
           class_embedding, W_q, b_q, W_k, b_k, W_g1, b_g1, W_g2, b_g2):
    r2 = lambda b: b.reshape(1, -1)

    wqk = jnp.concatenate([W_q, W_k], axis=1)
    h = pl.pallas_call(
        _graph_kernel,
        out_shape=jax.ShapeDtypeStruct((C, H), jnp.float32),
    )(class_embedding, wqk, r2(b_q), r2(b_k),
      W_g1, r2(b_g1), W_g2, r2(b_g2))

    full = lambda shape: pl.BlockSpec(shape, lambda i: (0, 0))
    out = pl.pallas_call(
        _big_kernel,
        grid=(B // BM,),
        in_specs=[
            pl.BlockSpec((BM, D), lambda i: (i, 0)),
            full((D, H)), full((1, H)), full((H, C)), full((1, C)),
            full((D, H)), full((1, H)), full((C, H)),
        ],
        out_specs=pl.BlockSpec((BM, C), lambda i: (i, 0)),
        out_shape=jax.ShapeDtypeStruct((B, C), jnp.float32),
    )(x, W_i2c_1, r2(b_i2c_1), W_i2c_2, r2(b_i2c_2),
      W_proj, r2(b_proj), h)
    return out


# speedup vs baseline: 1.0540x; 1.0540x over previous
import jax
import jax.numpy as jnp
from jax.experimental import pallas as pl
from jax.experimental.pallas import tpu as pltpu

B = 4096
D = 2048
H = 1024
C = 26
K_NEIGHBORS = 8
BM = 512


def _graph_kernel(ce_ref, wq_ref, bq_ref, wk_ref, bk_ref,
                  wg1_ref, bg1_ref, wg2_ref, bg2_ref, h_ref):
    ce = ce_ref[:]
    q = jnp.dot(ce, wq_ref[:], preferred_element_type=jnp.float32) + bq_ref[:]
    kk = jnp.dot(ce, wk_ref[:], preferred_element_type=jnp.float32) + bk_ref[:]
    scores = jax.lax.dot_general(
        q, kk, (((1,), (1,)), ((), ())),
        preferred_element_type=jnp.float32)

    col = jax.lax.broadcasted_iota(jnp.int32, (C, C), 1)
    big = jnp.int32(2 * C)
    work = scores
    sel = jnp.zeros((C, C), jnp.float32)
    for _ in range(K_NEIGHBORS):
        m = jnp.max(work, axis=1, keepdims=True)
        is_max = work >= m
        first = jnp.min(jnp.where(is_max, col, big), axis=1, keepdims=True)
        sel = sel + (col == first).astype(jnp.float32)
        work = jnp.where(col == first, -jnp.inf, work)

    eyef = (jax.lax.broadcasted_iota(jnp.int32, (C, C), 0) == col)
    eyef = eyef.astype(jnp.float32)
    mo = sel * (1.0 - eyef)
    deg = jnp.sum(mo, axis=0) + 1.0
    dinv = jax.lax.rsqrt(deg)

    def norm_matmul(y):
        yd = y * dinv[:, None]
        moty = jax.lax.dot_general(
            mo, yd, (((0,), (0,)), ((), ())),
            preferred_element_type=jnp.float32)
        return (moty + yd) * dinv[:, None]

    g1 = norm_matmul(
        jnp.dot(ce, wg1_ref[:], preferred_element_type=jnp.float32))
    g1 = jnp.maximum(g1 + bg1_ref[:], 0.0)
    h = norm_matmul(
        jnp.dot(g1, wg2_ref[:], preferred_element_type=jnp.float32))
    h_ref[:] = h + bg2_ref[:]


def _big_kernel(x_ref, w1_ref, b1_ref, w2_ref, b2_ref,
                wp_ref, bp_ref, h_ref, out_ref):
    xb = x_ref[:]
    u = jnp.dot(xb, w1_ref[:], preferred_element_type=jnp.float32)
    u = jnp.maximum(u + b1_ref[:], 0.0)
    p = jnp.dot(xb, wp_ref[:], preferred_element_type=jnp.float32)
    p = jnp.maximum(p + bp_ref[:], 0.0)
    cnn = jnp.dot(u, w2_ref[:], preferred_element_type=jnp.float32)
    gcn = jax.lax.dot_general(
        p, h_ref[:], (((1,), (1,)), ((), ())),
        preferred_element_type=jnp.float32)
    out_ref[:] = cnn + gcn + b2_ref[:]


@jax.jit
def kernel(x, W_i2c_1, b_i2c_1, W_i2c_2, b_i2c_2, W_proj, b_proj,
           class_embedding, W_q, b_q, W_k, b_k, W_g1, b_g1, W_g2, b_g2):
    r2 = lambda b: b.reshape(1, -1)

    h = pl.pallas_call(
        _graph_kernel,
        out_shape=jax.ShapeDtypeStruct((C, H), jnp.float32),
    )(class_embedding, W_q, r2(b_q), W_k, r2(b_k),
      W_g1, r2(b_g1), W_g2, r2(b_g2))

    full = lambda shape: pl.BlockSpec(shape, lambda i: (0, 0))
    out = pl.pallas_call(
        _big_kernel,
        grid=(B // BM,),
        in_specs=[
            pl.BlockSpec((BM, D), lambda i: (i, 0)),
            full((D, H)), full((1, H)), full((H, C)), full((1, C)),
            full((D, H)), full((1, H)), full((C, H)),
        ],
        out_specs=pl.BlockSpec((BM, C), lambda i: (i, 0)),
        out_shape=jax.ShapeDtypeStruct((B, C), jnp.float32),
    )(x, W_i2c_1, r2(b_i2c_1), W_i2c_2, r2(b_i2c_2),
      W_proj, r2(b_proj), h)
    return out
